# baseline (speedup 1.0000x reference)
import jax
import jax.numpy as jnp
from jax.experimental import pallas as pl
from jax.experimental.pallas import tpu as pltpu

_NC = 512
_HI = jax.lax.Precision.HIGHEST

_OMAP = {0: (-1, 1), 1: (0, 0), 2: (0, 1), 3: (1, 0)}
_DEC = {0: ((1, 0), (3, -1)), 1: ((0, 1), (2, 0))}

_SH9 = [(a, b) for a in (-1, 0, 1) for b in (-1, 0, 1)]

_T2 = []
for _di in range(4):
    _oi, _pi = _OMAP[_di]
    for _dj in range(4):
        _oj, _pj = _OMAP[_dj]
        _T2.append(((_oi + 1) * 130 + (_oj + 1), (_pi * 2 + _pj) * 64))


def _enc1_body(a_ref, w_ref, b_ref, o_ref):
    acc = jnp.dot(a_ref[...], w_ref[...],
                  preferred_element_type=jnp.float32)
    o_ref[...] = jnp.maximum(acc + b_ref[...], 0.0).astype(jnp.bfloat16)


def _enc1(a, w, b):
    m = a.shape[0]
    tm = 8192
    return pl.pallas_call(
        _enc1_body,
        grid=(m // tm,),
        in_specs=[pl.BlockSpec((tm, 108), lambda i: (i, 0)),
                  pl.BlockSpec((108, 64), lambda i: (0, 0)),
                  pl.BlockSpec((1, 64), lambda i: (0, 0))],
        out_specs=pl.BlockSpec((tm, 64), lambda i: (i, 0)),
        out_shape=jax.ShapeDtypeStruct((m, 64), jnp.bfloat16),
    )(a, w, b)


def _enc2_body(x_hbm, w_ref, b_ref, o_ref, xs_ref, cat_ref, sem):
    b = pl.program_id(0)
    t = pl.program_id(1)
    cp = pltpu.make_async_copy(x_hbm.at[b, pl.ds(t * 2080, 2344), :],
                               xs_ref, sem)
    cp.start()
    cp.wait()
    for k, (off, c0) in enumerate(_T2):
        cat_ref[:, k * 64:(k + 1) * 64] = xs_ref[off:off + 2080, c0:c0 + 64]
    acc = jnp.dot(cat_ref[...], w_ref[...], preferred_element_type=jnp.float32)
    o_ref[0] = acc + b_ref[...]


def _enc2(xpad, w, b):
    bsz = xpad.shape[0]
    return pl.pallas_call(
        _enc2_body,
        grid=(bsz, 8),
        in_specs=[pl.BlockSpec(memory_space=pl.ANY),
                  pl.BlockSpec((1024, 64), lambda i, j: (0, 0)),
                  pl.BlockSpec((1, 64), lambda i, j: (0, 0))],
        out_specs=pl.BlockSpec((1, 2080, 64), lambda i, j: (i, j, 0)),
        out_shape=jax.ShapeDtypeStruct((bsz, 16640, 64), jnp.float32),
        scratch_shapes=[pltpu.VMEM((2344, 256), jnp.bfloat16),
                        pltpu.VMEM((2080, 1024), jnp.bfloat16),
                        pltpu.SemaphoreType.DMA],
    )(xpad, w, b)


def _vq_body(z_ref, c_ref, csq_ref, q_ref, idx_ref, sq_ref):
    t = pl.program_id(0)
    z = z_ref[...]
    c = c_ref[...]
    s = jax.lax.dot_general(z.astype(jnp.bfloat16), c.astype(jnp.bfloat16),
                            (((1,), (1,)), ((), ())),
                            preferred_element_type=jnp.float32)
    d = csq_ref[...] - 2.0 * s
    dmin = jnp.min(d, axis=1, keepdims=True)
    iota = jax.lax.broadcasted_iota(jnp.int32, d.shape, 1)
    idx = jnp.min(jnp.where(d == dmin, iota, _NC), axis=1)
    onehot = (iota == idx[:, None]).astype(jnp.float32)
    q = jnp.dot(onehot, c, preferred_element_type=jnp.float32, precision=_HI)
    rows = t * 2080 + jax.lax.broadcasted_iota(jnp.int32, (2080, 1), 0)
    valid = jnp.remainder(rows, 130) < 128
    q = jnp.where(valid, q, 0.0)
    q_ref[...] = q.astype(jnp.bfloat16)
    idx_ref[0, 0] = idx
    sq = jnp.sum(jnp.where(valid, (q - z) ** 2, 0.0))
    sq_ref[0, 0] = jnp.full((2080,), sq, jnp.float32)


def _vq(zf, codebook):
    m = zf.shape[0]
    tm = 2080
    g = m // tm
    csq = (codebook ** 2).sum(1).reshape(1, _NC)
    return pl.pallas_call(
        _vq_body,
        grid=(g,),
        in_specs=[pl.BlockSpec((tm, 64), lambda i: (i, 0)),
                  pl.BlockSpec((_NC, 64), lambda i: (0, 0)),
                  pl.BlockSpec((1, _NC), lambda i: (0, 0))],
        out_specs=[pl.BlockSpec((tm, 64), lambda i: (i, 0)),
                   pl.BlockSpec((1, 1, tm), lambda i: (i, 0, 0)),
                   pl.BlockSpec((1, 1, tm), lambda i: (i, 0, 0))],
        out_shape=[jax.ShapeDtypeStruct((m, 64), jnp.bfloat16),
                   jax.ShapeDtypeStruct((g, 1, tm), jnp.int32),
                   jax.ShapeDtypeStruct((g, 1, tm), jnp.float32)],
    )(zf, codebook, csq)


def _dec1_body(qf_ref, w_ref, b_ref, o_ref, p_scr):
    t = pl.program_id(1)
    x = qf_ref[0, pl.ds(t * 2080, 2472), :]
    p_scr[...] = jnp.dot(x, w_ref[...],
                         preferred_element_type=jnp.float32)
    for r in range(2):
        for s in range(2):
            acc = jnp.zeros((2080, 64), jnp.float32)
            for kh, a in _DEC[r]:
                for kw, bb in _DEC[s]:
                    off = (a + 2) * 130 + bb
                    c0 = (kh * 4 + kw) * 64
                    acc = acc + p_scr[off:off + 2080, c0:c0 + 64]
            o_ref[0, 2 * r + s] = jnp.maximum(acc + b_ref[...],
                                              0.0).astype(jnp.bfloat16)


def _dec1(qpad, w, b):
    bsz = qpad.shape[0]
    return pl.pallas_call(
        _dec1_body,
        grid=(bsz, 8),
        in_specs=[pl.BlockSpec((1, 17160, 64), lambda i, j: (i, 0, 0)),
                  pl.BlockSpec((64, 1024), lambda i, j: (0, 0)),
                  pl.BlockSpec((1, 64), lambda i, j: (0, 0))],
        out_specs=pl.BlockSpec((1, 4, 2080, 64), lambda i, j: (i, 0, j, 0)),
        out_shape=jax.ShapeDtypeStruct((bsz, 4, 16640, 64), jnp.bfloat16),
        scratch_shapes=[pltpu.VMEM((2472, 1024), jnp.float32)],
    )(qpad, w, b)


def _dec2_body(hf_hbm, w_ref, b_ref, o_ref, xs_ref, cat_ref, sem):
    b = pl.program_id(0)
    t = pl.program_id(1)
    cp = pltpu.make_async_copy(hf_hbm.at[b, pl.ds(t * 4128, 4648), :],
                               xs_ref, sem)
    cp.start()
    cp.wait()
    for g, (a, bb) in enumerate(_SH9):
        off = 259 + a * 258 + bb
        cat_ref[:, g * 64:(g + 1) * 64] = xs_ref[off:off + 4128, :]
    acc = jnp.dot(cat_ref[...], w_ref[...], preferred_element_type=jnp.float32)
    o_ref[0] = jnp.tanh(acc + b_ref[...])


def _dec2(hpad, w, b):
    bsz = hpad.shape[0]
    return pl.pallas_call(
        _dec2_body,
        grid=(bsz, 16),
        in_specs=[pl.BlockSpec(memory_space=pl.ANY),
                  pl.BlockSpec((576, 32), lambda i, j: (0, 0)),
                  pl.BlockSpec((1, 32), lambda i, j: (0, 0))],
        out_specs=pl.BlockSpec((1, 4128, 32), lambda i, j: (i, j, 0)),
        out_shape=jax.ShapeDtypeStruct((bsz, 66048, 32), jnp.float32),
        scratch_shapes=[pltpu.VMEM((4648, 64), jnp.bfloat16),
                        pltpu.VMEM((4128, 576), jnp.bfloat16),
                        pltpu.SemaphoreType.DMA],
    )(hpad, w, b)


def kernel(images, enc_w1, enc_b1, enc_w2, enc_b2, codebook,
           dec_w1, dec_b1, dec_w2, dec_b2):
    bsz = images.shape[0]
    x = jnp.where(jnp.min(images) >= 0, images * 2.0 - 1.0, images)

    xp = x.reshape(bsz, 3, 256, 2, 256, 2).transpose(0, 2, 4, 1, 3, 5)
    xp = xp.reshape(bsz, 256, 256, 12)
    xpad = jnp.pad(xp, ((0, 0), (1, 1), (1, 1), (0, 0))).astype(jnp.bfloat16)
    cols = [xpad[:, 1 + oi:257 + oi, 1 + oj:257 + oj, :] for oi, oj in _SH9]
    a1 = jnp.concatenate(cols, axis=-1).reshape(bsz * 65536, 108)
    w1_48 = jnp.transpose(enc_w1, (2, 3, 1, 0)).reshape(48, 64)
    row_map = [48] * 108
    for di in range(4):
        oi, pi = _OMAP[di]
        for dj in range(4):
            oj, pj = _OMAP[dj]
            g = (oi + 1) * 3 + (oj + 1)
            for c in range(3):
                row_map[g * 12 + c * 4 + pi * 2 + pj] = (di * 4 + dj) * 3 + c
    w1 = jnp.concatenate([w1_48, jnp.zeros((1, 64), w1_48.dtype)], axis=0)
    w1 = w1[jnp.array(row_map)].astype(jnp.bfloat16)
    h = _enc1(a1, w1, enc_b1.reshape(1, 64))

    hp = h.reshape(bsz, 128, 2, 128, 2, 64).transpose(0, 1, 3, 2, 4, 5)
    hp = hp.reshape(bsz, 128, 128, 256)
    hpad = jnp.pad(hp, ((0, 0), (1, 3), (1, 1), (0, 0))).reshape(bsz, 17160, 256)
    w2 = jnp.transpose(enc_w2, (2, 3, 1, 0)).reshape(1024, 64).astype(jnp.bfloat16)
    lat = _enc2(hpad, w2, enc_b2.reshape(1, 64))
    zf = lat.reshape(bsz * 16640, 64)

    q, idxo, sqo = _vq(zf, codebook)
    puzzles = idxo.reshape(bsz, 128, 130)[:, :, :128]
    vq_loss = 1.25 * jnp.sum(sqo[:, 0, 0]) / (bsz * 16384 * 64)

    qim = q.reshape(bsz, 128, 130, 64)
    qpad = jnp.pad(qim, ((0, 0), (2, 2), (0, 0), (0, 0))).reshape(bsz, 17160, 64)
    w1d = jnp.transpose(dec_w1, (0, 2, 3, 1)).reshape(64, 1024).astype(jnp.bfloat16)
    planes1 = _dec1(qpad, w1d, dec_b1.reshape(1, 64))
    h2 = planes1.reshape(bsz, 2, 2, 128, 130, 64)[:, :, :, :, :128, :]
    h2 = h2.transpose(0, 3, 1, 4, 2, 5).reshape(bsz, 256, 256, 64)

    h2pad = jnp.pad(h2, ((0, 0), (1, 3), (1, 1), (0, 0))).reshape(bsz, 67080, 64)
    blocks = []
    for a, bb in _SH9:
        for r in range(2):
            for s in range(2):
                kh = dict((aa, kk) for kk, aa in _DEC[r]).get(a)
                kw = dict((aa, kk) for kk, aa in _DEC[s]).get(bb)
                if kh is None or kw is None:
                    blocks.append(jnp.zeros((64, 8), dec_w2.dtype))
                else:
                    blocks.append(jnp.pad(dec_w2[:, :, kh, kw],
                                          ((0, 0), (0, 5))))
    w5 = jnp.concatenate([jnp.concatenate(blocks[4 * g:4 * g + 4], axis=1)
                          for g in range(9)], axis=0).astype(jnp.bfloat16)
    b2d = jnp.tile(jnp.pad(dec_b2, (0, 5)), 4).reshape(1, 32)
    planes2 = _dec2(h2pad, w5, b2d)
    rec6 = planes2.reshape(bsz, 256, 258, 2, 2, 8)[:, :, :256, :, :, :3]
    recon = rec6.transpose(0, 5, 1, 3, 2, 4).reshape(bsz, 3, 512, 512)

    return (recon, puzzles, vq_loss)


# speedup vs baseline: 1.3429x; 1.3429x over previous
import jax
import jax.numpy as jnp
from jax.experimental import pallas as pl
from jax.experimental.pallas import tpu as pltpu

_NC = 512
_HI = jax.lax.Precision.HIGHEST

_OMAP = {0: (-1, 1), 1: (0, 0), 2: (0, 1), 3: (1, 0)}
_DEC = {0: ((1, 0), (3, -1)), 1: ((0, 1), (2, 0))}

_SH9 = [(a, b) for a in (-1, 0, 1) for b in (-1, 0, 1)]

_T2 = []
for _di in range(4):
    _oi, _pi = _OMAP[_di]
    for _dj in range(4):
        _oj, _pj = _OMAP[_dj]
        _T2.append(((_oi + 1) * 130 + (_oj + 1), (_pi * 2 + _pj) * 64))


def _enc1_body(a_ref, w_ref, b_ref, o_ref):
    acc = jnp.dot(a_ref[...], w_ref[...],
                  preferred_element_type=jnp.float32)
    o_ref[...] = jnp.maximum(acc + b_ref[...], 0.0).astype(jnp.bfloat16)


def _enc1(a, w, b):
    m = a.shape[0]
    tm = 8192
    return pl.pallas_call(
        _enc1_body,
        grid=(m // tm,),
        in_specs=[pl.BlockSpec((tm, 108), lambda i: (i, 0)),
                  pl.BlockSpec((108, 64), lambda i: (0, 0)),
                  pl.BlockSpec((1, 64), lambda i: (0, 0))],
        out_specs=pl.BlockSpec((tm, 64), lambda i: (i, 0)),
        out_shape=jax.ShapeDtypeStruct((m, 64), jnp.bfloat16),
    )(a, w, b)


def _enc2_body(x_hbm, w_ref, b_ref, o_ref, xs_ref, cat_ref, sem):
    b = pl.program_id(0)
    t = pl.program_id(1)
    cp = pltpu.make_async_copy(x_hbm.at[b, pl.ds(t * 2080, 2344), :],
                               xs_ref, sem)
    cp.start()
    cp.wait()
    for k, (off, c0) in enumerate(_T2):
        cat_ref[:, k * 64:(k + 1) * 64] = xs_ref[off:off + 2080, c0:c0 + 64]
    acc = jnp.dot(cat_ref[...], w_ref[...], preferred_element_type=jnp.float32)
    o_ref[0] = acc + b_ref[...]


def _enc2(xpad, w, b):
    bsz = xpad.shape[0]
    return pl.pallas_call(
        _enc2_body,
        grid=(bsz, 8),
        in_specs=[pl.BlockSpec(memory_space=pl.ANY),
                  pl.BlockSpec((1024, 64), lambda i, j: (0, 0)),
                  pl.BlockSpec((1, 64), lambda i, j: (0, 0))],
        out_specs=pl.BlockSpec((1, 2080, 64), lambda i, j: (i, j, 0)),
        out_shape=jax.ShapeDtypeStruct((bsz, 16640, 64), jnp.float32),
        scratch_shapes=[pltpu.VMEM((2344, 256), jnp.bfloat16),
                        pltpu.VMEM((2080, 1024), jnp.bfloat16),
                        pltpu.SemaphoreType.DMA],
    )(xpad, w, b)


def _vq_body(z_ref, c_ref, csq_ref, q_ref, idx_ref, sq_ref):
    t = pl.program_id(0)
    z = z_ref[...]
    c = c_ref[...]
    s = jax.lax.dot_general(z.astype(jnp.bfloat16), c.astype(jnp.bfloat16),
                            (((1,), (1,)), ((), ())),
                            preferred_element_type=jnp.float32)
    d = csq_ref[...] - 2.0 * s
    dmin = jnp.min(d, axis=1, keepdims=True)
    iota = jax.lax.broadcasted_iota(jnp.int32, d.shape, 1)
    idx = jnp.min(jnp.where(d == dmin, iota, _NC), axis=1)
    onehot = (iota == idx[:, None]).astype(jnp.float32)
    q = jnp.dot(onehot, c, preferred_element_type=jnp.float32, precision=_HI)
    rows = t * 2080 + jax.lax.broadcasted_iota(jnp.int32, (2080, 1), 0)
    valid = jnp.remainder(rows, 130) < 128
    q = jnp.where(valid, q, 0.0)
    q_ref[...] = q.astype(jnp.bfloat16)
    idx_ref[0, 0] = idx
    sq = jnp.sum(jnp.where(valid, (q - z) ** 2, 0.0))
    sq_ref[0, 0] = jnp.full((2080,), sq, jnp.float32)


def _vq(zf, codebook):
    m = zf.shape[0]
    tm = 2080
    g = m // tm
    csq = (codebook ** 2).sum(1).reshape(1, _NC)
    return pl.pallas_call(
        _vq_body,
        grid=(g,),
        in_specs=[pl.BlockSpec((tm, 64), lambda i: (i, 0)),
                  pl.BlockSpec((_NC, 64), lambda i: (0, 0)),
                  pl.BlockSpec((1, _NC), lambda i: (0, 0))],
        out_specs=[pl.BlockSpec((tm, 64), lambda i: (i, 0)),
                   pl.BlockSpec((1, 1, tm), lambda i: (i, 0, 0)),
                   pl.BlockSpec((1, 1, tm), lambda i: (i, 0, 0))],
        out_shape=[jax.ShapeDtypeStruct((m, 64), jnp.bfloat16),
                   jax.ShapeDtypeStruct((g, 1, tm), jnp.int32),
                   jax.ShapeDtypeStruct((g, 1, tm), jnp.float32)],
    )(zf, codebook, csq)


def _dec1_body(qf_ref, w_ref, b_ref, o_ref, p_scr):
    t = pl.program_id(1)
    x = qf_ref[0, pl.ds(t * 2080, 2472), :]
    p_scr[...] = jnp.dot(x, w_ref[...],
                         preferred_element_type=jnp.float32)
    for r in range(2):
        for s in range(2):
            acc = jnp.zeros((2080, 64), jnp.float32)
            for kh, a in _DEC[r]:
                for kw, bb in _DEC[s]:
                    off = (a + 2) * 130 + bb
                    c0 = (kh * 4 + kw) * 64
                    acc = acc + p_scr[off:off + 2080, c0:c0 + 64]
            o_ref[0, 2 * r + s] = jnp.maximum(acc + b_ref[...],
                                              0.0).astype(jnp.bfloat16)


def _dec1(qpad, w, b):
    bsz = qpad.shape[0]
    return pl.pallas_call(
        _dec1_body,
        grid=(bsz, 8),
        in_specs=[pl.BlockSpec((1, 17160, 64), lambda i, j: (i, 0, 0)),
                  pl.BlockSpec((64, 1024), lambda i, j: (0, 0)),
                  pl.BlockSpec((1, 64), lambda i, j: (0, 0))],
        out_specs=pl.BlockSpec((1, 4, 2080, 64), lambda i, j: (i, 0, j, 0)),
        out_shape=jax.ShapeDtypeStruct((bsz, 4, 16640, 64), jnp.bfloat16),
        scratch_shapes=[pltpu.VMEM((2472, 1024), jnp.float32)],
    )(qpad, w, b)


def _dec2_body(hf_hbm, w_ref, b_ref, o_ref, xs_ref, cat_ref, sem):
    b = pl.program_id(0)
    t = pl.program_id(1)
    cp = pltpu.make_async_copy(hf_hbm.at[b, pl.ds(t * 4128, 4648), :],
                               xs_ref, sem)
    cp.start()
    cp.wait()
    for g, (a, bb) in enumerate(_SH9):
        off = 259 + a * 258 + bb
        cat_ref[:, g * 64:(g + 1) * 64] = xs_ref[off:off + 4128, :]
    acc = jnp.dot(cat_ref[...], w_ref[...], preferred_element_type=jnp.float32)
    v = jnp.tanh(acc + b_ref[...])
    v = v.reshape(16, 258, 2, 2, 8)[:, :256, :, :, :3]
    o_ref[0] = jnp.transpose(v, (4, 0, 2, 1, 3)).reshape(3, 32, 512)


def _dec2(hpad, w, b):
    bsz = hpad.shape[0]
    return pl.pallas_call(
        _dec2_body,
        grid=(bsz, 16),
        in_specs=[pl.BlockSpec(memory_space=pl.ANY),
                  pl.BlockSpec((576, 32), lambda i, j: (0, 0)),
                  pl.BlockSpec((1, 32), lambda i, j: (0, 0))],
        out_specs=pl.BlockSpec((1, 3, 32, 512), lambda i, j: (i, 0, j, 0)),
        out_shape=jax.ShapeDtypeStruct((bsz, 3, 512, 512), jnp.float32),
        scratch_shapes=[pltpu.VMEM((4648, 64), jnp.bfloat16),
                        pltpu.VMEM((4128, 576), jnp.bfloat16),
                        pltpu.SemaphoreType.DMA],
    )(hpad, w, b)


def kernel(images, enc_w1, enc_b1, enc_w2, enc_b2, codebook,
           dec_w1, dec_b1, dec_w2, dec_b2):
    bsz = images.shape[0]
    x = jnp.where(jnp.min(images) >= 0, images * 2.0 - 1.0, images)

    xp = x.reshape(bsz, 3, 256, 2, 256, 2).transpose(0, 2, 4, 1, 3, 5)
    xp = xp.reshape(bsz, 256, 256, 12)
    xpad = jnp.pad(xp, ((0, 0), (1, 1), (1, 1), (0, 0))).astype(jnp.bfloat16)
    cols = [xpad[:, 1 + oi:257 + oi, 1 + oj:257 + oj, :] for oi, oj in _SH9]
    a1 = jnp.concatenate(cols, axis=-1).reshape(bsz * 65536, 108)
    w1_48 = jnp.transpose(enc_w1, (2, 3, 1, 0)).reshape(48, 64)
    row_map = [48] * 108
    for di in range(4):
        oi, pi = _OMAP[di]
        for dj in range(4):
            oj, pj = _OMAP[dj]
            g = (oi + 1) * 3 + (oj + 1)
            for c in range(3):
                row_map[g * 12 + c * 4 + pi * 2 + pj] = (di * 4 + dj) * 3 + c
    w1 = jnp.concatenate([w1_48, jnp.zeros((1, 64), w1_48.dtype)], axis=0)
    w1 = w1[jnp.array(row_map)].astype(jnp.bfloat16)
    h = _enc1(a1, w1, enc_b1.reshape(1, 64))

    hp = h.reshape(bsz, 128, 2, 128, 2, 64).transpose(0, 1, 3, 2, 4, 5)
    hp = hp.reshape(bsz, 128, 128, 256)
    hpad = jnp.pad(hp, ((0, 0), (1, 3), (1, 1), (0, 0))).reshape(bsz, 17160, 256)
    w2 = jnp.transpose(enc_w2, (2, 3, 1, 0)).reshape(1024, 64).astype(jnp.bfloat16)
    lat = _enc2(hpad, w2, enc_b2.reshape(1, 64))
    zf = lat.reshape(bsz * 16640, 64)

    q, idxo, sqo = _vq(zf, codebook)
    puzzles = idxo.reshape(bsz, 128, 130)[:, :, :128]
    vq_loss = 1.25 * jnp.sum(sqo[:, 0, 0]) / (bsz * 16384 * 64)

    qim = q.reshape(bsz, 128, 130, 64)
    qpad = jnp.pad(qim, ((0, 0), (2, 2), (0, 0), (0, 0))).reshape(bsz, 17160, 64)
    w1d = jnp.transpose(dec_w1, (0, 2, 3, 1)).reshape(64, 1024).astype(jnp.bfloat16)
    planes1 = _dec1(qpad, w1d, dec_b1.reshape(1, 64))
    h2 = planes1.reshape(bsz, 2, 2, 128, 130, 64)[:, :, :, :, :128, :]
    h2 = h2.transpose(0, 3, 1, 4, 2, 5).reshape(bsz, 256, 256, 64)

    h2pad = jnp.pad(h2, ((0, 0), (1, 3), (1, 1), (0, 0))).reshape(bsz, 67080, 64)
    blocks = []
    for a, bb in _SH9:
        for r in range(2):
            for s in range(2):
                kh = dict((aa, kk) for kk, aa in _DEC[r]).get(a)
                kw = dict((aa, kk) for kk, aa in _DEC[s]).get(bb)
                if kh is None or kw is None:
                    blocks.append(jnp.zeros((64, 8), dec_w2.dtype))
                else:
                    blocks.append(jnp.pad(dec_w2[:, :, kh, kw],
                                          ((0, 0), (0, 5))))
    w5 = jnp.concatenate([jnp.concatenate(blocks[4 * g:4 * g + 4], axis=1)
                          for g in range(9)], axis=0).astype(jnp.bfloat16)
    b2d = jnp.tile(jnp.pad(dec_b2, (0, 5)), 4).reshape(1, 32)
    recon = _dec2(h2pad, w5, b2d)

    return (recon, puzzles, vq_loss)


# speedup vs baseline: 1.4660x; 1.0916x over previous
import jax
import jax.numpy as jnp
from jax.experimental import pallas as pl
from jax.experimental.pallas import tpu as pltpu

_NC = 512
_HI = jax.lax.Precision.HIGHEST

_OMAP = {0: (-1, 1), 1: (0, 0), 2: (0, 1), 3: (1, 0)}
_DEC = {0: ((1, 0), (3, -1)), 1: ((0, 1), (2, 0))}

_SH9 = [(a, b) for a in (-1, 0, 1) for b in (-1, 0, 1)]

_T2 = []
for _di in range(4):
    _oi, _pi = _OMAP[_di]
    for _dj in range(4):
        _oj, _pj = _OMAP[_dj]
        _T2.append(((_oi + 1) * 130 + (_oj + 1), (_pi * 2 + _pj) * 64))


def _enc1_body(a_ref, w_ref, b_ref, o_ref):
    acc = jnp.dot(a_ref[...], w_ref[...],
                  preferred_element_type=jnp.float32)
    o_ref[...] = jnp.maximum(acc + b_ref[...], 0.0).astype(jnp.bfloat16)


def _enc1(a, w, b):
    m = a.shape[0]
    tm = 8192
    return pl.pallas_call(
        _enc1_body,
        grid=(m // tm,),
        in_specs=[pl.BlockSpec((tm, 108), lambda i: (i, 0)),
                  pl.BlockSpec((108, 64), lambda i: (0, 0)),
                  pl.BlockSpec((1, 64), lambda i: (0, 0))],
        out_specs=pl.BlockSpec((tm, 64), lambda i: (i, 0)),
        out_shape=jax.ShapeDtypeStruct((m, 64), jnp.bfloat16),
    )(a, w, b)


def _enc2_body(x_hbm, w_ref, b_ref, o_ref, xs_ref, cat_ref, sem):
    b = pl.program_id(0)
    t = pl.program_id(1)
    cp = pltpu.make_async_copy(x_hbm.at[b, pl.ds(t * 2080, 2344), :],
                               xs_ref, sem)
    cp.start()
    cp.wait()
    for k, (off, c0) in enumerate(_T2):
        cat_ref[:, k * 64:(k + 1) * 64] = xs_ref[off:off + 2080, c0:c0 + 64]
    acc = jnp.dot(cat_ref[...], w_ref[...], preferred_element_type=jnp.float32)
    o_ref[0] = acc + b_ref[...]


def _enc2(xpad, w, b):
    bsz = xpad.shape[0]
    return pl.pallas_call(
        _enc2_body,
        grid=(bsz, 8),
        in_specs=[pl.BlockSpec(memory_space=pl.ANY),
                  pl.BlockSpec((1024, 64), lambda i, j: (0, 0)),
                  pl.BlockSpec((1, 64), lambda i, j: (0, 0))],
        out_specs=pl.BlockSpec((1, 2080, 64), lambda i, j: (i, j, 0)),
        out_shape=jax.ShapeDtypeStruct((bsz, 16640, 64), jnp.float32),
        scratch_shapes=[pltpu.VMEM((2344, 256), jnp.bfloat16),
                        pltpu.VMEM((2080, 1024), jnp.bfloat16),
                        pltpu.SemaphoreType.DMA],
    )(xpad, w, b)


def _vq_body(z_ref, c_ref, csq_ref, q_ref, idx_ref, sq_ref):
    t = pl.program_id(0)
    z = z_ref[...]
    c = c_ref[...]
    s = jax.lax.dot_general(z.astype(jnp.bfloat16), c.astype(jnp.bfloat16),
                            (((1,), (1,)), ((), ())),
                            preferred_element_type=jnp.float32)
    d = csq_ref[...] - 2.0 * s
    dmin = jnp.min(d, axis=1, keepdims=True)
    iota = jax.lax.broadcasted_iota(jnp.int32, d.shape, 1)
    idx = jnp.min(jnp.where(d == dmin, iota, _NC), axis=1)
    onehot = (iota == idx[:, None]).astype(jnp.float32)
    q = jnp.dot(onehot, c, preferred_element_type=jnp.float32, precision=_HI)
    rows = t * 2080 + jax.lax.broadcasted_iota(jnp.int32, (2080, 1), 0)
    valid = jnp.remainder(rows, 130) < 128
    q = jnp.where(valid, q, 0.0)
    q_ref[...] = q.astype(jnp.bfloat16)
    idx_ref[0, 0] = idx
    sq = jnp.sum(jnp.where(valid, (q - z) ** 2, 0.0))
    sq_ref[0, 0] = jnp.full((2080,), sq, jnp.float32)


def _vq(zf, codebook):
    m = zf.shape[0]
    tm = 2080
    g = m // tm
    csq = (codebook ** 2).sum(1).reshape(1, _NC)
    return pl.pallas_call(
        _vq_body,
        grid=(g,),
        in_specs=[pl.BlockSpec((tm, 64), lambda i: (i, 0)),
                  pl.BlockSpec((_NC, 64), lambda i: (0, 0)),
                  pl.BlockSpec((1, _NC), lambda i: (0, 0))],
        out_specs=[pl.BlockSpec((tm, 64), lambda i: (i, 0)),
                   pl.BlockSpec((1, 1, tm), lambda i: (i, 0, 0)),
                   pl.BlockSpec((1, 1, tm), lambda i: (i, 0, 0))],
        out_shape=[jax.ShapeDtypeStruct((m, 64), jnp.bfloat16),
                   jax.ShapeDtypeStruct((g, 1, tm), jnp.int32),
                   jax.ShapeDtypeStruct((g, 1, tm), jnp.float32)],
    )(zf, codebook, csq)


def _dec1_body(qf_ref, w_ref, b_ref, o_ref, p_scr):
    t = pl.program_id(1)

    @pl.when(jnp.logical_or(t == 0, t == 9))
    def _pad():
        o_ref[0] = jnp.zeros((8256, 64), jnp.bfloat16)

    @pl.when(jnp.logical_and(t >= 1, t <= 8))
    def _compute():
        tt = t - 1
        x = qf_ref[0, pl.ds(tt * 2080, 2472), :]
        p_scr[...] = jnp.dot(x, w_ref[...],
                             preferred_element_type=jnp.float32)
        accs = []
        for r in range(2):
            for s in range(2):
                acc = jnp.zeros((2080, 64), jnp.float32)
                for kh, a in _DEC[r]:
                    for kw, bb in _DEC[s]:
                        off = (a + 2) * 130 + bb
                        c0 = (kh * 4 + kw) * 64
                        acc = acc + p_scr[off:off + 2080, c0:c0 + 64]
                accs.append(jnp.maximum(acc + b_ref[...], 0.0))
        v = jnp.concatenate(accs, axis=1)
        v = v.reshape(16, 130, 2, 2, 64)[:, :128]
        v = jnp.transpose(v, (0, 2, 1, 3, 4)).reshape(32, 256, 64)
        v = jnp.pad(v, ((0, 0), (1, 1), (0, 0)))
        o_ref[0] = v.reshape(8256, 64).astype(jnp.bfloat16)


def _dec1(qpad, w, b):
    bsz = qpad.shape[0]
    return pl.pallas_call(
        _dec1_body,
        grid=(bsz, 10),
        in_specs=[pl.BlockSpec((1, 17160, 64), lambda i, j: (i, 0, 0)),
                  pl.BlockSpec((64, 1024), lambda i, j: (0, 0)),
                  pl.BlockSpec((1, 64), lambda i, j: (0, 0))],
        out_specs=pl.BlockSpec((1, 8256, 64), lambda i, j: (i, j, 0)),
        out_shape=jax.ShapeDtypeStruct((bsz, 82560, 64), jnp.bfloat16),
        scratch_shapes=[pltpu.VMEM((2472, 1024), jnp.float32)],
    )(qpad, w, b)


def _dec2_body(hf_hbm, w_ref, b_ref, o_ref, xs_ref, cat_ref, sem):
    b = pl.program_id(0)
    t = pl.program_id(1)
    cp = pltpu.make_async_copy(hf_hbm.at[b, pl.ds(t * 4128 + 7992, 4656), :],
                               xs_ref, sem)
    cp.start()
    cp.wait()
    for g, (a, bb) in enumerate(_SH9):
        off = 265 + a * 258 + bb
        cat_ref[:, g * 64:(g + 1) * 64] = xs_ref[off:off + 4128, :]
    acc = jnp.dot(cat_ref[...], w_ref[...], preferred_element_type=jnp.float32)
    v = jnp.tanh(acc + b_ref[...])
    v = v.reshape(16, 258, 2, 2, 8)[:, :256, :, :, :3]
    o_ref[0] = jnp.transpose(v, (4, 0, 2, 1, 3)).reshape(3, 32, 512)


def _dec2(hpad, w, b):
    bsz = hpad.shape[0]
    return pl.pallas_call(
        _dec2_body,
        grid=(bsz, 16),
        in_specs=[pl.BlockSpec(memory_space=pl.ANY),
                  pl.BlockSpec((576, 32), lambda i, j: (0, 0)),
                  pl.BlockSpec((1, 32), lambda i, j: (0, 0))],
        out_specs=pl.BlockSpec((1, 3, 32, 512), lambda i, j: (i, 0, j, 0)),
        out_shape=jax.ShapeDtypeStruct((bsz, 3, 512, 512), jnp.float32),
        scratch_shapes=[pltpu.VMEM((4656, 64), jnp.bfloat16),
                        pltpu.VMEM((4128, 576), jnp.bfloat16),
                        pltpu.SemaphoreType.DMA],
    )(hpad, w, b)


def kernel(images, enc_w1, enc_b1, enc_w2, enc_b2, codebook,
           dec_w1, dec_b1, dec_w2, dec_b2):
    bsz = images.shape[0]
    x = jnp.where(jnp.min(images) >= 0, images * 2.0 - 1.0, images)

    xp = x.reshape(bsz, 3, 256, 2, 256, 2).transpose(0, 2, 4, 1, 3, 5)
    xp = xp.reshape(bsz, 256, 256, 12)
    xpad = jnp.pad(xp, ((0, 0), (1, 1), (1, 1), (0, 0))).astype(jnp.bfloat16)
    cols = [xpad[:, 1 + oi:257 + oi, 1 + oj:257 + oj, :] for oi, oj in _SH9]
    a1 = jnp.concatenate(cols, axis=-1).reshape(bsz * 65536, 108)
    w1_48 = jnp.transpose(enc_w1, (2, 3, 1, 0)).reshape(48, 64)
    row_map = [48] * 108
    for di in range(4):
        oi, pi = _OMAP[di]
        for dj in range(4):
            oj, pj = _OMAP[dj]
            g = (oi + 1) * 3 + (oj + 1)
            for c in range(3):
                row_map[g * 12 + c * 4 + pi * 2 + pj] = (di * 4 + dj) * 3 + c
    w1 = jnp.concatenate([w1_48, jnp.zeros((1, 64), w1_48.dtype)], axis=0)
    w1 = w1[jnp.array(row_map)].astype(jnp.bfloat16)
    h = _enc1(a1, w1, enc_b1.reshape(1, 64))

    hp = h.reshape(bsz, 128, 2, 128, 2, 64).transpose(0, 1, 3, 2, 4, 5)
    hp = hp.reshape(bsz, 128, 128, 256)
    hpad = jnp.pad(hp, ((0, 0), (1, 3), (1, 1), (0, 0))).reshape(bsz, 17160, 256)
    w2 = jnp.transpose(enc_w2, (2, 3, 1, 0)).reshape(1024, 64).astype(jnp.bfloat16)
    lat = _enc2(hpad, w2, enc_b2.reshape(1, 64))
    zf = lat.reshape(bsz * 16640, 64)

    q, idxo, sqo = _vq(zf, codebook)
    puzzles = idxo.reshape(bsz, 128, 130)[:, :, :128]
    vq_loss = 1.25 * jnp.sum(sqo[:, 0, 0]) / (bsz * 16384 * 64)

    qim = q.reshape(bsz, 128, 130, 64)
    qpad = jnp.pad(qim, ((0, 0), (2, 2), (0, 0), (0, 0))).reshape(bsz, 17160, 64)
    w1d = jnp.transpose(dec_w1, (0, 2, 3, 1)).reshape(64, 1024).astype(jnp.bfloat16)
    h2pad = _dec1(qpad, w1d, dec_b1.reshape(1, 64))

    blocks = []
    for a, bb in _SH9:
        for r in range(2):
            for s in range(2):
                kh = dict((aa, kk) for kk, aa in _DEC[r]).get(a)
                kw = dict((aa, kk) for kk, aa in _DEC[s]).get(bb)
                if kh is None or kw is None:
                    blocks.append(jnp.zeros((64, 8), dec_w2.dtype))
                else:
                    blocks.append(jnp.pad(dec_w2[:, :, kh, kw],
                                          ((0, 0), (0, 5))))
    w5 = jnp.concatenate([jnp.concatenate(blocks[4 * g:4 * g + 4], axis=1)
                          for g in range(9)], axis=0).astype(jnp.bfloat16)
    b2d = jnp.tile(jnp.pad(dec_b2, (0, 5)), 4).reshape(1, 32)
    recon = _dec2(h2pad, w5, b2d)

    return (recon, puzzles, vq_loss)
